# Initial kernel scaffold; baseline (speedup 1.0000x reference)
import jax
import jax.numpy as jnp
from jax.experimental import pallas as pl


def kernel(label, box, label_table, bbox_table, W, b):
    raise NotImplementedError("write your pallas kernel here")


# speedup vs baseline: 1.1894x; 1.1894x over previous
import functools

import jax
import jax.numpy as jnp
from jax import lax
from jax.experimental import pallas as pl
from jax.experimental.pallas import tpu as pltpu
from jax.experimental.pallas import tpu_sc as plsc

LANES = 16
NW = 32
CHUNK = 16
LBL_ROWS = 40
T_ROWS = LBL_ROWS + 4 * 128


def _proj_table_kernel(e_ref, wt_ref, b_ref, t_ref):
    t = jnp.dot(e_ref[...], wt_ref[...], preferred_element_type=jnp.float32)
    row = lax.broadcasted_iota(jnp.int32, t.shape, 0)
    t_ref[...] = t + jnp.where(row < LBL_ROWS, 1.0, 0.0) * b_ref[...]


def _build_proj_table(label_table, bbox_table, W, b):
    d_model = W.shape[0]
    e = jnp.zeros((T_ROWS, 5 * 128), dtype=jnp.float32)
    e = e.at[: label_table.shape[0], 0:128].set(label_table)
    for k in range(4):
        e = e.at[LBL_ROWS + 128 * k : LBL_ROWS + 128 * (k + 1),
                 128 * (k + 1) : 128 * (k + 2)].set(bbox_table)
    wt = W.T
    return pl.pallas_call(
        _proj_table_kernel,
        out_shape=jax.ShapeDtypeStruct((T_ROWS, d_model), jnp.float32),
    )(e, wt, b.reshape(1, d_model))


def _sc_gather_sum(tbl, label_flat, box_flat, tokens, d_model):
    per_worker = tokens // NW
    iters = per_worker // CHUNK
    mesh = plsc.VectorSubcoreMesh(core_axis_name="c", subcore_axis_name="s")

    @functools.partial(
        pl.kernel,
        out_type=jax.ShapeDtypeStruct((tokens, d_model), jnp.float32),
        mesh=mesh,
        scratch_types=[
            pltpu.VMEM((5 * CHUNK,), jnp.int32),
            pltpu.VMEM((5 * CHUNK, d_model), jnp.float32),
            pltpu.VMEM((CHUNK, d_model), jnp.float32),
            pltpu.SemaphoreType.DMA,
        ],
    )
    def k(tbl_hbm, label_hbm, box_hbm, out_hbm, idx_v, g_v, out_v, sem):
        wid = lax.axis_index("s") * 2 + lax.axis_index("c")
        w_base = wid * per_worker

        def body(it, carry):
            base = w_base + it * CHUNK
            pltpu.sync_copy(label_hbm.at[pl.ds(base, CHUNK)],
                            idx_v.at[pl.ds(0, CHUNK)])
            for j in range(4):
                pltpu.sync_copy(box_hbm.at[pl.ds(j * tokens + base, CHUNK)],
                                idx_v.at[pl.ds((j + 1) * CHUNK, CHUNK)])
            for j in range(4):
                sl = pl.ds((j + 1) * CHUNK, LANES)
                idx_v[sl] = idx_v[sl] + (LBL_ROWS + 128 * j)
            pltpu.async_copy(tbl_hbm.at[idx_v], g_v, sem).wait()

            def sum_body(i, c2):
                c = i // (d_model // LANES)
                h = (i % (d_model // LANES)) * LANES
                acc = g_v[c, pl.ds(h, LANES)]
                for j in range(1, 5):
                    acc = acc + g_v[j * CHUNK + c, pl.ds(h, LANES)]
                out_v[c, pl.ds(h, LANES)] = acc
                return c2

            lax.fori_loop(0, CHUNK * (d_model // LANES), sum_body, 0,
                          unroll=4)
            pltpu.sync_copy(out_v, out_hbm.at[pl.ds(base, CHUNK)])
            return carry

        lax.fori_loop(0, iters, body, 0)

    return k(tbl, label_flat, box_flat)


def kernel(label, box, label_table, bbox_table, W, b):
    s, n = label.shape
    d_model = W.shape[0]
    tokens = s * n
    tbl = _build_proj_table(label_table, bbox_table, W, b)
    label_flat = label.reshape(tokens).astype(jnp.int32)
    box_flat = box.reshape(tokens, 4).astype(jnp.int32).T.reshape(4 * tokens)
    out = _sc_gather_sum(tbl, label_flat, box_flat, tokens, d_model)
    return out.reshape(s, n, d_model)
